# baseline (speedup 1.0000x reference)
import jax
import jax.numpy as jnp
from jax.experimental import pallas as pl


def _softmax_body(x_ref, o_ref):
    x = x_ref[...]
    m = jnp.max(x, axis=-1, keepdims=True)
    e = jnp.exp(x - m)
    s = jnp.sum(e, axis=-1, keepdims=True)
    o_ref[...] = e / s


def kernel(pred_obj_logits, pred_verb_logits, pred_sub_boxes, pred_obj_boxes, target_sizes):
    K = 100
    B, Q, C = pred_obj_logits.shape
    V = pred_verb_logits.shape[-1]
    CP = 128
    x = jnp.pad(pred_obj_logits, ((0, 0), (0, 0), (0, CP - C)), constant_values=-1e30)

    probs = pl.pallas_call(
        _softmax_body,
        grid=(B,),
        in_specs=[pl.BlockSpec((1, Q, CP), lambda b: (b, 0, 0))],
        out_specs=pl.BlockSpec((1, Q, CP), lambda b: (b, 0, 0)),
        out_shape=jax.ShapeDtypeStruct((B, Q, CP), jnp.float32),
    )(x)

    obj_prob = probs[:, :, :C]
    verb_scores = jax.nn.sigmoid(pred_verb_logits)
    flat = obj_prob.reshape(B, -1)
    topk_values, topk_indexes = jax.lax.top_k(flat, K)
    obj_scores = topk_values
    topk_boxes = topk_indexes // C
    obj_labels = topk_indexes % C
    vs = jnp.take_along_axis(
        verb_scores, jnp.broadcast_to(topk_boxes[:, :, None], (B, K, V)), axis=1)
    out_obj_boxes = jnp.take_along_axis(
        pred_obj_boxes, jnp.broadcast_to(topk_boxes[:, :, None], (B, K, 4)), axis=1)
    out_sub_boxes = jnp.take_along_axis(
        pred_sub_boxes, jnp.broadcast_to(topk_boxes[:, :, None], (B, K, 4)), axis=1)
    img_h = target_sizes[:, 0].astype(jnp.float32)
    img_w = target_sizes[:, 1].astype(jnp.float32)
    scale_fct = jnp.stack([img_w, img_h, img_w, img_h], axis=1)

    def box_cxcywh_to_xyxy(b):
        cx, cy, w, h = b[..., 0], b[..., 1], b[..., 2], b[..., 3]
        return jnp.stack([cx - 0.5 * w, cy - 0.5 * h, cx + 0.5 * w, cy + 0.5 * h], axis=-1)

    sub_boxes = box_cxcywh_to_xyxy(out_sub_boxes) * scale_fct[:, None, :]
    obj_boxes = box_cxcywh_to_xyxy(out_obj_boxes) * scale_fct[:, None, :]
    sl = jnp.full_like(obj_labels, 0)
    labels = jnp.concatenate([sl, obj_labels], axis=1)
    boxes = jnp.concatenate([sub_boxes, obj_boxes], axis=1)
    verb_scores_out = vs * obj_scores[:, :, None]
    ids = jnp.arange(2 * K, dtype=jnp.int32)
    return (labels, boxes, verb_scores_out, ids[:K], ids[K:])
